# baseline (speedup 1.0000x reference)
import functools

import jax
import jax.numpy as jnp
from jax import lax
from jax.experimental import pallas as pl
from jax.experimental.pallas import tpu as pltpu
from jax.experimental.pallas import tpu_sc as plsc

_NUM_TILES = 32
_LANES = 16
_CHUNK = 128


@functools.lru_cache(maxsize=None)
def _build(B, D, L):
    assert B == _NUM_TILES and D % _CHUNK == 0

    mesh = plsc.VectorSubcoreMesh(core_axis_name="c", subcore_axis_name="s")

    def body(xa_hbm, xb_hbm, idxoff_hbm, out_a_hbm, out_b_hbm,
             idxoff_v, ibuf_a, ibuf_b, buf_a, buf_b, sem):
        b = lax.axis_index("s") * 2 + lax.axis_index("c")

        pltpu.sync_copy(idxoff_hbm, idxoff_v)
        pair = idxoff_v[pl.ds(2 * b, _LANES)]
        col_a = pair[0]
        col_b = col_a + pair[1]

        base = b * (D * L)
        iota = lax.iota(jnp.int32, _LANES)
        for j in range(D // _LANES):
            dterm = base + (j * _LANES + iota) * L
            ibuf_a[pl.ds(j * _LANES, _LANES)] = dterm + col_a
            ibuf_b[pl.ds(j * _LANES, _LANES)] = dterm + col_b

        copies = []
        for j in range(D // _CHUNK):
            sl = pl.ds(j * _CHUNK, _CHUNK)
            copies.append(
                pltpu.async_copy(xa_hbm.at[ibuf_a.at[sl]], buf_a.at[sl], sem))
            copies.append(
                pltpu.async_copy(xb_hbm.at[ibuf_b.at[sl]], buf_b.at[sl], sem))
        for c in copies:
            c.wait()

        pltpu.sync_copy(buf_a, out_a_hbm.at[b])
        pltpu.sync_copy(buf_b, out_b_hbm.at[b])

    return pl.kernel(
        body,
        out_type=(
            jax.ShapeDtypeStruct((B, D), jnp.float32),
            jax.ShapeDtypeStruct((B, D), jnp.float32),
        ),
        mesh=mesh,
        scratch_types=[
            pltpu.VMEM((2 * B + _LANES,), jnp.int32),
            pltpu.VMEM((D,), jnp.int32),
            pltpu.VMEM((D,), jnp.int32),
            pltpu.VMEM((D,), jnp.float32),
            pltpu.VMEM((D,), jnp.float32),
            pltpu.SemaphoreType.DMA,
        ],
    )


def kernel(x_a, x_b):
    b, d, l = x_a.shape
    max_window = l // 2
    key = jax.random.key(1)
    k1, k2 = jax.random.split(key)
    idx = jax.random.randint(k1, (b,), 0, l - max_window)
    offset = jax.random.randint(k2, (b,), 1, max_window)
    idxoff = jnp.stack([idx, offset], axis=1).reshape(-1).astype(jnp.int32)
    idxoff = jnp.pad(idxoff, (0, _LANES))

    gather = _build(b, d, l)
    return gather(x_a.reshape(-1), x_b.reshape(-1), idxoff)


# speedup vs baseline: 7.1496x; 7.1496x over previous
import functools

import jax
import jax.numpy as jnp
from jax import lax
from jax.experimental import pallas as pl
from jax.experimental.pallas import tpu as pltpu
from jax.experimental.pallas import tpu_sc as plsc

_NUM_TILES = 32
_LANES = 16
_LANE_TILE = 128
_CH = 256


@functools.lru_cache(maxsize=None)
def _build(B, D, L):
    assert B == _NUM_TILES and D % _CH == 0
    n_chunks = D // _CH

    mesh = plsc.VectorSubcoreMesh(core_axis_name="c", subcore_axis_name="s")

    def body(xa_hbm, xb_hbm, idxoff_hbm, out_a_hbm, out_b_hbm,
             idxoff_v, buf0, buf1, obuf_a, obuf_b, sem):
        b = lax.axis_index("s") * 2 + lax.axis_index("c")

        pltpu.sync_copy(idxoff_hbm, idxoff_v)
        pair = idxoff_v[pl.ds(2 * b, _LANES)]
        col_a = pair[0]
        col_b = col_a + pair[1]

        bufs = (buf0, buf1)
        iota = lax.iota(jnp.int32, _LANES)
        steps = []
        for x_hbm, obuf, col in ((xa_hbm, obuf_a, col_a),
                                 (xb_hbm, obuf_b, col_b)):
            win = (col // _LANE_TILE) * _LANE_TILE
            lane = col - win
            for h in range(n_chunks):
                steps.append((x_hbm, obuf, win, lane, h))

        def start(i):
            x_hbm, _, win, _, h = steps[i]
            return pltpu.async_copy(
                x_hbm.at[b, pl.ds(h * _CH, _CH), pl.ds(win, _LANE_TILE)],
                bufs[i % 2], sem)

        pending = start(0)
        for i in range(len(steps)):
            nxt = start(i + 1) if i + 1 < len(steps) else None
            pending.wait()
            _, obuf, _, lane, h = steps[i]
            lane_v = jnp.broadcast_to(lane, (_LANES,))
            buf = bufs[i % 2]
            for j in range(_CH // _LANES):
                rows = j * _LANES + iota
                v = plsc.load_gather(buf, [rows, lane_v])
                obuf[pl.ds(h * _CH + j * _LANES, _LANES)] = v
            if nxt is not None:
                pending = nxt

        pltpu.sync_copy(obuf_a, out_a_hbm.at[b])
        pltpu.sync_copy(obuf_b, out_b_hbm.at[b])

    return pl.kernel(
        body,
        out_type=(
            jax.ShapeDtypeStruct((B, D), jnp.float32),
            jax.ShapeDtypeStruct((B, D), jnp.float32),
        ),
        mesh=mesh,
        compiler_params=pltpu.CompilerParams(needs_layout_passes=False),
        scratch_types=[
            pltpu.VMEM((2 * B + _LANES,), jnp.int32),
            pltpu.VMEM((_CH, _LANE_TILE), jnp.float32),
            pltpu.VMEM((_CH, _LANE_TILE), jnp.float32),
            pltpu.VMEM((D,), jnp.float32),
            pltpu.VMEM((D,), jnp.float32),
            pltpu.SemaphoreType.DMA,
        ],
    )


def kernel(x_a, x_b):
    b, d, l = x_a.shape
    max_window = l // 2
    key = jax.random.key(1)
    k1, k2 = jax.random.split(key)
    idx = jax.random.randint(k1, (b,), 0, l - max_window)
    offset = jax.random.randint(k2, (b,), 1, max_window)
    idxoff = jnp.stack([idx, offset], axis=1).reshape(-1).astype(jnp.int32)
    idxoff = jnp.pad(idxoff, (0, _LANES))

    gather = _build(b, d, l)
    return gather(x_a, x_b, idxoff)


# speedup vs baseline: 11.4448x; 1.6008x over previous
import functools

import jax
import jax.numpy as jnp
from jax import lax
from jax.experimental import pallas as pl
from jax.experimental.pallas import tpu as pltpu
from jax.experimental.pallas import tpu_sc as plsc

_NUM_TILES = 32
_LANES = 16
_LANE_TILE = 128
_CH = 256


@functools.lru_cache(maxsize=None)
def _build(B, D, L):
    assert B == _NUM_TILES and D % _CH == 0
    n_chunks = D // _CH

    mesh = plsc.VectorSubcoreMesh(core_axis_name="c", subcore_axis_name="s")

    def body(xa_hbm, xb_hbm, idxoff_hbm, out_a_hbm, out_b_hbm,
             idxoff_v, buf0, buf1, obuf_a, obuf_b, sem):
        b = lax.axis_index("s") * 2 + lax.axis_index("c")

        pltpu.sync_copy(idxoff_hbm, idxoff_v)
        pair = idxoff_v[pl.ds(2 * b, _LANES)]
        col_a = pair[0]
        col_b = col_a + pair[1]

        bufs = (buf0, buf1)
        iota = lax.iota(jnp.int32, _LANES)
        steps = []
        for x_hbm, obuf, col in ((xa_hbm, obuf_a, col_a),
                                 (xb_hbm, obuf_b, col_b)):
            win = (col // _LANE_TILE) * _LANE_TILE
            lane = col - win
            for h in range(n_chunks):
                steps.append((x_hbm, obuf, win, lane, h))

        def start(i):
            x_hbm, _, win, _, h = steps[i]
            return pltpu.async_copy(
                x_hbm.at[b, pl.ds(h * _CH, _CH), pl.ds(win, _LANE_TILE)],
                bufs[i % 2], sem)

        pending = start(0)
        for i in range(len(steps)):
            nxt = start(i + 1) if i + 1 < len(steps) else None
            pending.wait()
            _, obuf, _, lane, h = steps[i]
            lane_v = jnp.broadcast_to(lane, (_LANES,))
            buf = bufs[i % 2]
            for j in range(_CH // _LANES):
                rows = j * _LANES + iota
                v = plsc.load_gather(buf, [rows, lane_v])
                obuf[pl.ds(h * _CH + j * _LANES, _LANES)] = v
            if nxt is not None:
                pending = nxt

        pltpu.sync_copy(obuf_a, out_a_hbm.at[b])
        pltpu.sync_copy(obuf_b, out_b_hbm.at[b])

    return pl.kernel(
        body,
        out_type=(
            jax.ShapeDtypeStruct((B, D), jnp.float32),
            jax.ShapeDtypeStruct((B, D), jnp.float32),
        ),
        mesh=mesh,
        compiler_params=pltpu.CompilerParams(needs_layout_passes=False),
        scratch_types=[
            pltpu.VMEM((2 * B + _LANES,), jnp.int32),
            pltpu.VMEM((_CH, _LANE_TILE), jnp.float32),
            pltpu.VMEM((_CH, _LANE_TILE), jnp.float32),
            pltpu.VMEM((D,), jnp.float32),
            pltpu.VMEM((D,), jnp.float32),
            pltpu.SemaphoreType.DMA,
        ],
    )


def kernel(x_a, x_b):
    b, d, l = x_a.shape
    with jax.ensure_compile_time_eval():
        max_window = l // 2
        key = jax.random.key(1)
        k1, k2 = jax.random.split(key)
        idx = jax.random.randint(k1, (b,), 0, l - max_window)
        offset = jax.random.randint(k2, (b,), 1, max_window)
        idxoff = jnp.stack([idx, offset], axis=1).reshape(-1)
        idxoff = jnp.pad(idxoff, (0, _LANES)).astype(jnp.int32)

    gather = _build(b, d, l)
    return gather(x_a, x_b, idxoff)
